# baseline (speedup 1.0000x reference)
import functools

import jax
import jax.numpy as jnp
from jax import lax
from jax.experimental import pallas as pl
from jax.experimental.pallas import tpu as pltpu
from jax.experimental.pallas import tpu_sc as plsc

N = 10000
E = 320000
DIN = 128
H = 64
NC = 2
NS = 16
NW = NC * NS
EPW = E // NW
CH = 80
NB = 5
N_PAD = 10240
RPT = N_PAD // NS
RBLK = 2048
GRID = N_PAD // RBLK

_f32 = jnp.float32


def _proj_body(xm, xf, Wm, bm, Wf, bf, hm, hf):
    hm[...] = jnp.dot(xm[...], Wm[...], preferred_element_type=_f32) + bm[...]
    hf[...] = jnp.dot(xf[...], Wf[...], preferred_element_type=_f32) + bf[...]


def _proj(x_mch, x_mft, W_mch, b_mch, W_mft, b_mft):
    return pl.pallas_call(
        _proj_body,
        grid=(GRID,),
        in_specs=[
            pl.BlockSpec((RBLK, DIN), lambda i: (i, 0)),
            pl.BlockSpec((RBLK, DIN), lambda i: (i, 0)),
            pl.BlockSpec((DIN, H), lambda i: (0, 0)),
            pl.BlockSpec((1, H), lambda i: (0, 0)),
            pl.BlockSpec((DIN, H), lambda i: (0, 0)),
            pl.BlockSpec((1, H), lambda i: (0, 0)),
        ],
        out_specs=[pl.BlockSpec((RBLK, H), lambda i: (i, 0))] * 2,
        out_shape=[jax.ShapeDtypeStruct((N, H), _f32)] * 2,
    )(x_mch, x_mft, W_mch, b_mch, W_mft, b_mft)


TPC = (E // NS) // CH


def _make_agg(with_counts):
    mesh = plsc.VectorSubcoreMesh(core_axis_name="c", subcore_axis_name="s")
    out_type = [
        jax.ShapeDtypeStruct((N_PAD, H), _f32),
        jax.ShapeDtypeStruct((N_PAD, H), _f32),
    ]
    scratch = [
        pltpu.VMEM((TPC // 2, CH), jnp.int32),
        pltpu.VMEM((TPC // 2, CH), jnp.int32),
        pltpu.VMEM((2, NB, CH, H), _f32),
        pltpu.VMEM_SHARED((N_PAD, H), _f32),
        pltpu.SemaphoreType.DMA,
        pltpu.SemaphoreType.DMA,
    ]
    if with_counts:
        out_type += [
            jax.ShapeDtypeStruct((N,), _f32),
            jax.ShapeDtypeStruct((N,), _f32),
        ]
        scratch += [
            pltpu.VMEM_SHARED((N,), _f32),
            pltpu.VMEM((128,), _f32),
            pltpu.SemaphoreType.DMA,
        ]

    def body(hm, hf, srcr, dstr, z2d, *rest):
        if with_counts:
            (z1d, ones_h, o_aggf, o_aggm, o_cntf, o_cntm,
             src_v, dst_v, rows, acc, sem_g, sem_s,
             cnt, ones_v, sem_c) = rest
        else:
            (o_aggf, o_aggm,
             src_v, dst_v, rows, acc, sem_g, sem_s) = rest
        c = lax.axis_index("c")
        s = lax.axis_index("s")
        rs = pl.ds(s * RPT, RPT)
        pltpu.sync_copy(z2d, acc.at[rs])
        if with_counts:
            @pl.when(s == 0)
            def _():
                pltpu.sync_copy(z1d, cnt)

            pltpu.sync_copy(ones_h, ones_v)
        plsc.subcore_barrier()

        def sweep_half(tab, gidx, sidx):
            G = (TPC // 2) // NB

            def fire_g(g, bank):
                for b in range(NB):
                    pltpu.async_copy(tab.at[gidx.at[g * NB + b]],
                                     rows.at[bank, b], sem_g)

            def drain_g(g, bank):
                for b in range(NB):
                    pltpu.make_async_copy(tab.at[gidx.at[g * NB + b]],
                                          rows.at[bank, b], sem_g).wait()

            def fire_s(g, bank):
                for b in range(NB):
                    di = sidx.at[g * NB + b]
                    pltpu.async_copy(rows.at[bank, b], acc.at[di], sem_s,
                                     add=True)
                    if with_counts:
                        pltpu.async_copy(ones_v.at[pl.ds(0, CH)], cnt.at[di],
                                         sem_c, add=True)

            def drain_s(g, bank):
                for b in range(NB):
                    di = sidx.at[g * NB + b]
                    pltpu.make_async_copy(rows.at[bank, b], acc.at[di],
                                          sem_s).wait()
                    if with_counts:
                        pltpu.make_async_copy(ones_v.at[pl.ds(0, CH)],
                                              cnt.at[di], sem_c).wait()

            fire_g(0, 0)

            def step(g, carry):
                bank = lax.rem(g, 2)
                obank = 1 - bank

                @pl.when(g > 0)
                def _():
                    drain_s(g - 1, obank)

                @pl.when(g + 1 < G)
                def _():
                    fire_g(g + 1, obank)

                drain_g(g, bank)
                fire_s(g, bank)
                return carry

            lax.fori_loop(0, G, step, 0)
            drain_s(G - 1, (G - 1) % 2)

        for half in range(2):
            hs = pl.ds(half * (TPC // 2), TPC // 2)
            pltpu.sync_copy(srcr.at[s, hs], src_v)
            pltpu.sync_copy(dstr.at[s, hs], dst_v)

            @pl.when(c == 0)
            def _():
                sweep_half(hm, src_v, dst_v)

            @pl.when(c == 1)
            def _():
                sweep_half(hf, dst_v, src_v)

        plsc.subcore_barrier()

        @pl.when(c == 0)
        def _():
            pltpu.sync_copy(acc.at[rs], o_aggf.at[rs])

        @pl.when(c == 1)
        def _():
            pltpu.sync_copy(acc.at[rs], o_aggm.at[rs])

        if with_counts:
            @pl.when((c == 0) & (s == 0))
            def _():
                pltpu.sync_copy(cnt, o_cntf)

            @pl.when((c == 1) & (s == 0))
            def _():
                pltpu.sync_copy(cnt, o_cntm)

    return pl.kernel(
        body, out_type=out_type, mesh=mesh, scratch_types=scratch,
        compiler_params=pltpu.CompilerParams(use_tc_tiling_on_sc=False, needs_layout_passes=False))


_agg_with_counts = _make_agg(True)
_agg_plain = _make_agg(False)


def _sage_body(relu, aggf, cntf, hf, Wlf, blf, Wrf,
               aggm, cntm, hm, Wlm, blm, Wrm, of, om):
    def side(agg, cnt, h, Wl, bl, Wr):
        mean = agg / jnp.maximum(cnt, 1.0)[:, None]
        r = (jnp.dot(mean, Wl[...], preferred_element_type=_f32) + bl[...]
             + jnp.dot(h[...], Wr[...], preferred_element_type=_f32))
        return jnp.maximum(r, 0.0) if relu else r

    of[...] = side(aggf[...], cntf[...], hf, Wlf, blf, Wrf)
    om[...] = side(aggm[...], cntm[...], hm, Wlm, blm, Wrm)


def _sage_layer(relu, aggf, cntf, hf, Wlf, blf, Wrf, aggm, cntm, hm, Wlm, blm, Wrm):
    agg_spec = pl.BlockSpec((RBLK, H), lambda i: (i, 0))
    cnt_spec = pl.BlockSpec((RBLK,), lambda i: (i,))
    h_spec = pl.BlockSpec((RBLK, H), lambda i: (i, 0))
    w_spec = pl.BlockSpec((H, H), lambda i: (0, 0))
    b_spec = pl.BlockSpec((1, H), lambda i: (0, 0))
    return pl.pallas_call(
        functools.partial(_sage_body, relu),
        grid=(GRID,),
        in_specs=[agg_spec, cnt_spec, h_spec, w_spec, b_spec, w_spec,
                  agg_spec, cnt_spec, h_spec, w_spec, b_spec, w_spec],
        out_specs=[h_spec, h_spec],
        out_shape=[jax.ShapeDtypeStruct((N, H), _f32)] * 2,
    )(aggf, cntf, hf, Wlf, blf, Wrf, aggm, cntm, hm, Wlm, blm, Wrm)


def _final_body(aggf, cntf, hf, Wlf, blf, Wrf,
                aggm, cntm, hm, Wlm, blm, Wrm,
                We1t, We1b, be1, oA, oB):
    def side(agg, cnt, h, Wl, bl, Wr):
        mean = agg / jnp.maximum(cnt, 1.0)[:, None]
        return (jnp.dot(mean, Wl[...], preferred_element_type=_f32) + bl[...]
                + jnp.dot(h[...], Wr[...], preferred_element_type=_f32))

    h_mft2 = side(aggf[...], cntf[...], hf, Wlf, blf, Wrf)
    h_mch2 = side(aggm[...], cntm[...], hm, Wlm, blm, Wrm)
    oA[...] = (jnp.dot(h_mch2, We1t[...], preferred_element_type=_f32)
               + be1[...]).astype(jnp.bfloat16)
    oB[...] = jnp.dot(h_mft2, We1b[...],
                      preferred_element_type=_f32).astype(jnp.bfloat16)


def _final_proj(aggf, cntf, hf, Wlf, blf, Wrf, aggm, cntm, hm, Wlm, blm, Wrm,
                We1t, We1b, be1):
    agg_spec = pl.BlockSpec((RBLK, H), lambda i: (i, 0))
    cnt_spec = pl.BlockSpec((RBLK,), lambda i: (i,))
    h_spec = pl.BlockSpec((RBLK, H), lambda i: (i, 0))
    w_spec = pl.BlockSpec((H, H), lambda i: (0, 0))
    b_spec = pl.BlockSpec((1, H), lambda i: (0, 0))
    return pl.pallas_call(
        _final_body,
        grid=(GRID,),
        in_specs=[agg_spec, cnt_spec, h_spec, w_spec, b_spec, w_spec,
                  agg_spec, cnt_spec, h_spec, w_spec, b_spec, w_spec,
                  w_spec, w_spec, b_spec],
        out_specs=[h_spec, h_spec],
        out_shape=[jax.ShapeDtypeStruct((N, H), jnp.bfloat16)] * 2,
    )(aggf, cntf, hf, Wlf, blf, Wrf, aggm, cntm, hm, Wlm, blm, Wrm,
      We1t, We1b, be1)


CHF = 80
NCHF = EPW // CHF


def _make_edge_scorer():
    mesh = plsc.VectorSubcoreMesh(core_axis_name="c", subcore_axis_name="s")
    out_type = jax.ShapeDtypeStruct((NW, EPW), _f32)
    scratch = [
        pltpu.VMEM((NCHF, CHF), jnp.int32),
        pltpu.VMEM((NCHF, CHF), jnp.int32),
        pltpu.VMEM((2, CHF, H), jnp.bfloat16),
        pltpu.VMEM((2, CHF, H), jnp.bfloat16),
        pltpu.VMEM((H,), jnp.bfloat16),
        pltpu.VMEM((16,), _f32),
        pltpu.VMEM((EPW,), _f32),
        pltpu.SemaphoreType.DMA,
        pltpu.SemaphoreType.DMA,
    ]

    def body(a_h, b_h, srcr, dstr, w_h, be2_h, out_h,
             src_v, dst_v, a_rows, b_rows, w_v, be2_v, out_v, sem_a, sem_b):
        c = lax.axis_index("c")
        s = lax.axis_index("s")
        hs = pl.ds(c * NCHF, NCHF)
        pltpu.sync_copy(srcr.at[s, hs], src_v)
        pltpu.sync_copy(dstr.at[s, hs], dst_v)
        pltpu.sync_copy(w_h, w_v)
        pltpu.sync_copy(be2_h, be2_v)
        wq = []
        for q in range(H // 32):
            w0, w1 = plsc.unpack(w_v[pl.ds(q * 32, 32)],
                                 format=plsc.PackFormat.INTERLEAVED,
                                 preferred_element_type=_f32)
            wq.append((w0, w1))
        zero16 = jnp.zeros((16,), _f32)
        zero32 = jnp.zeros((32,), jnp.bfloat16)
        lane = lax.iota(jnp.int32, 16)

        def fire(j, bank):
            ca = pltpu.async_copy(a_h.at[src_v.at[j]], a_rows.at[bank], sem_a)
            cb = pltpu.async_copy(b_h.at[dst_v.at[j]], b_rows.at[bank], sem_b)
            return ca, cb

        fire(0, 0)

        def chunk(j, carry):
            bank = lax.rem(j, 2)

            @pl.when(j + 1 < NCHF)
            def _():
                fire(j + 1, lax.rem(j + 1, 2))

            pltpu.make_async_copy(a_h.at[src_v.at[j]], a_rows.at[bank],
                                  sem_a).wait()
            pltpu.make_async_copy(b_h.at[dst_v.at[j]], b_rows.at[bank],
                                  sem_b).wait()

            def group(g, carry2):
                vec = zero16
                for e in range(16):
                    idx = g * 16 + e
                    acc = zero16
                    for q in range(H // 32):
                        a0, a1 = plsc.unpack(
                            a_rows[bank, idx, pl.ds(q * 32, 32)],
                            format=plsc.PackFormat.INTERLEAVED,
                            preferred_element_type=_f32)
                        b0, b1 = plsc.unpack(
                            b_rows[bank, idx, pl.ds(q * 32, 32)],
                            format=plsc.PackFormat.INTERLEAVED,
                            preferred_element_type=_f32)
                        s0 = jnp.maximum(a0 + b0, 0.0)
                        s1 = jnp.maximum(a1 + b1, 0.0)
                        acc = acc + s0 * wq[q][0] + s1 * wq[q][1]
                    vec = jnp.where(lane == e, jnp.sum(acc), vec)
                out_v[pl.ds(j * CHF + g * 16, 16)] = vec
                return carry2

            lax.fori_loop(0, CHF // 16, group, 0)
            return carry

        lax.fori_loop(0, NCHF, chunk, 0)
        bev = be2_v[...]

        def sig(i, carry):
            ix = pl.ds(i * 16, 16)
            v = out_v[ix] + bev
            out_v[ix] = 1.0 / (1.0 + jnp.exp(-v))
            return carry

        lax.fori_loop(0, EPW // 16, sig, 0)
        pltpu.sync_copy(out_v, out_h.at[2 * s + c])

    return pl.kernel(
        body, out_type=out_type, mesh=mesh, scratch_types=scratch,
        compiler_params=pltpu.CompilerParams(use_tc_tiling_on_sc=False, needs_layout_passes=False))


_edge_scorer = _make_edge_scorer()


def kernel(x_mch, x_mft, edge_index, W_mch, b_mch, W_mft, b_mft,
           Wl1_m2f, bl1_m2f, Wr1_m2f, Wl1_f2m, bl1_f2m, Wr1_f2m,
           Wl2_m2f, bl2_m2f, Wr2_m2f, Wl2_f2m, bl2_f2m, Wr2_f2m,
           We1, be1, We2, be2):
    src = edge_index[0].astype(jnp.int32)
    dst = edge_index[1].astype(jnp.int32)
    srcr = src.reshape(NS, TPC, CH)
    dstr = dst.reshape(NS, TPC, CH)
    z2d = jnp.zeros((RPT, H), _f32)
    z1d = jnp.zeros((N,), _f32)
    ones_h = jnp.ones((128,), _f32)

    h_mch, h_mft = _proj(x_mch, x_mft, W_mch, b_mch.reshape(1, H),
                         W_mft, b_mft.reshape(1, H))

    aggf1, aggm1, cntf, cntm = _agg_with_counts(
        h_mch, h_mft, srcr, dstr, z2d, z1d, ones_h)

    h_mft1, h_mch1 = _sage_layer(
        True,
        aggf1, cntf, h_mft, Wl1_m2f, bl1_m2f.reshape(1, H), Wr1_m2f,
        aggm1, cntm, h_mch, Wl1_f2m, bl1_f2m.reshape(1, H), Wr1_f2m)

    aggf2, aggm2 = _agg_plain(h_mch1, h_mft1, srcr, dstr, z2d)

    Aproj, Bproj = _final_proj(
        aggf2, cntf, h_mft1, Wl2_m2f, bl2_m2f.reshape(1, H), Wr2_m2f,
        aggm2, cntm, h_mch1, Wl2_f2m, bl2_f2m.reshape(1, H), Wr2_f2m,
        We1[:H], We1[H:], be1.reshape(1, H))

    out = _edge_scorer(Aproj, Bproj, srcr, dstr,
                       We2[:, 0].astype(jnp.bfloat16),
                       jnp.broadcast_to(be2, (16,)).astype(_f32))
    return out.reshape(E, 1)


# speedup vs baseline: 1.0339x; 1.0339x over previous
import functools

import jax
import jax.numpy as jnp
from jax import lax
from jax.experimental import pallas as pl
from jax.experimental.pallas import tpu as pltpu
from jax.experimental.pallas import tpu_sc as plsc

N = 10000
E = 320000
DIN = 128
H = 64
NC = 2
NS = 16
NW = NC * NS
EPW = E // NW
CH = 80
NB = 5
N_PAD = 10240
RPT = N_PAD // NS
RBLK = 2048
GRID = N_PAD // RBLK

_f32 = jnp.float32


def _proj_body(xm, xf, Wm, bm, Wf, bf, hm, hf):
    hm[...] = jnp.dot(xm[...], Wm[...], preferred_element_type=_f32) + bm[...]
    hf[...] = jnp.dot(xf[...], Wf[...], preferred_element_type=_f32) + bf[...]


def _proj(x_mch, x_mft, W_mch, b_mch, W_mft, b_mft):
    return pl.pallas_call(
        _proj_body,
        grid=(GRID,),
        in_specs=[
            pl.BlockSpec((RBLK, DIN), lambda i: (i, 0)),
            pl.BlockSpec((RBLK, DIN), lambda i: (i, 0)),
            pl.BlockSpec((DIN, H), lambda i: (0, 0)),
            pl.BlockSpec((1, H), lambda i: (0, 0)),
            pl.BlockSpec((DIN, H), lambda i: (0, 0)),
            pl.BlockSpec((1, H), lambda i: (0, 0)),
        ],
        out_specs=[pl.BlockSpec((RBLK, H), lambda i: (i, 0))] * 2,
        out_shape=[jax.ShapeDtypeStruct((N, H), _f32)] * 2,
    )(x_mch, x_mft, W_mch, b_mch, W_mft, b_mft)


TPC = (E // NS) // CH


def _make_agg(with_counts):
    mesh = plsc.VectorSubcoreMesh(core_axis_name="c", subcore_axis_name="s")
    out_type = [
        jax.ShapeDtypeStruct((N_PAD, H), _f32),
        jax.ShapeDtypeStruct((N_PAD, H), _f32),
    ]
    scratch = [
        pltpu.VMEM((TPC // 2, CH), jnp.int32),
        pltpu.VMEM((TPC // 2, CH), jnp.int32),
        pltpu.VMEM((2, NB, CH, H), _f32),
        pltpu.VMEM_SHARED((N_PAD, H), _f32),
        pltpu.SemaphoreType.DMA,
        pltpu.SemaphoreType.DMA,
    ]
    if with_counts:
        out_type += [
            jax.ShapeDtypeStruct((N,), _f32),
            jax.ShapeDtypeStruct((N,), _f32),
        ]
        scratch += [
            pltpu.VMEM_SHARED((N,), _f32),
            pltpu.VMEM((128,), _f32),
            pltpu.SemaphoreType.DMA,
        ]

    def body(hm, hf, ei, z2d, *rest):
        if with_counts:
            (z1d, ones_h, o_aggf, o_aggm, o_cntf, o_cntm,
             src_v, dst_v, rows, acc, sem_g, sem_s,
             cnt, ones_v, sem_c) = rest
        else:
            (o_aggf, o_aggm,
             src_v, dst_v, rows, acc, sem_g, sem_s) = rest
        c = lax.axis_index("c")
        s = lax.axis_index("s")
        rs = pl.ds(s * RPT, RPT)
        pltpu.sync_copy(z2d, acc.at[rs])
        if with_counts:
            @pl.when(s == 0)
            def _():
                pltpu.sync_copy(z1d, cnt)

            pltpu.sync_copy(ones_h, ones_v)
        plsc.subcore_barrier()

        def sweep_half(tab, gidx, sidx):
            G = (TPC // 2) // NB

            def fire_g(g, bank):
                for b in range(NB):
                    pltpu.async_copy(tab.at[gidx.at[g * NB + b]],
                                     rows.at[bank, b], sem_g)

            def drain_g(g, bank):
                for b in range(NB):
                    pltpu.make_async_copy(tab.at[gidx.at[g * NB + b]],
                                          rows.at[bank, b], sem_g).wait()

            def fire_s(g, bank):
                for b in range(NB):
                    di = sidx.at[g * NB + b]
                    pltpu.async_copy(rows.at[bank, b], acc.at[di], sem_s,
                                     add=True)
                    if with_counts:
                        pltpu.async_copy(ones_v.at[pl.ds(0, CH)], cnt.at[di],
                                         sem_c, add=True)

            def drain_s(g, bank):
                for b in range(NB):
                    di = sidx.at[g * NB + b]
                    pltpu.make_async_copy(rows.at[bank, b], acc.at[di],
                                          sem_s).wait()
                    if with_counts:
                        pltpu.make_async_copy(ones_v.at[pl.ds(0, CH)],
                                              cnt.at[di], sem_c).wait()

            fire_g(0, 0)

            def step(g, carry):
                bank = lax.rem(g, 2)
                obank = 1 - bank

                @pl.when(g > 0)
                def _():
                    drain_s(g - 1, obank)

                @pl.when(g + 1 < G)
                def _():
                    fire_g(g + 1, obank)

                drain_g(g, bank)
                fire_s(g, bank)
                return carry

            lax.fori_loop(0, G, step, 0)
            drain_s(G - 1, (G - 1) % 2)

        for half in range(2):
            hs = pl.ds(half * (TPC // 2), TPC // 2)
            pltpu.sync_copy(ei.at[0, s, hs], src_v)
            pltpu.sync_copy(ei.at[1, s, hs], dst_v)

            @pl.when(c == 0)
            def _():
                sweep_half(hm, src_v, dst_v)

            @pl.when(c == 1)
            def _():
                sweep_half(hf, dst_v, src_v)

        plsc.subcore_barrier()

        @pl.when(c == 0)
        def _():
            pltpu.sync_copy(acc.at[rs], o_aggf.at[rs])

        @pl.when(c == 1)
        def _():
            pltpu.sync_copy(acc.at[rs], o_aggm.at[rs])

        if with_counts:
            @pl.when((c == 0) & (s == 0))
            def _():
                pltpu.sync_copy(cnt, o_cntf)

            @pl.when((c == 1) & (s == 0))
            def _():
                pltpu.sync_copy(cnt, o_cntm)

    return pl.kernel(
        body, out_type=out_type, mesh=mesh, scratch_types=scratch,
        compiler_params=pltpu.CompilerParams(use_tc_tiling_on_sc=False, needs_layout_passes=False))


_agg_with_counts = _make_agg(True)
_agg_plain = _make_agg(False)


def _sage_body(relu, aggf, cntf, hf, Wlf, blf, Wrf,
               aggm, cntm, hm, Wlm, blm, Wrm, of, om):
    def side(agg, cnt, h, Wl, bl, Wr):
        mean = agg / jnp.maximum(cnt, 1.0)[:, None]
        r = (jnp.dot(mean, Wl[...], preferred_element_type=_f32) + bl[...]
             + jnp.dot(h[...], Wr[...], preferred_element_type=_f32))
        return jnp.maximum(r, 0.0) if relu else r

    of[...] = side(aggf[...], cntf[...], hf, Wlf, blf, Wrf)
    om[...] = side(aggm[...], cntm[...], hm, Wlm, blm, Wrm)


def _sage_layer(relu, aggf, cntf, hf, Wlf, blf, Wrf, aggm, cntm, hm, Wlm, blm, Wrm):
    agg_spec = pl.BlockSpec((RBLK, H), lambda i: (i, 0))
    cnt_spec = pl.BlockSpec((RBLK,), lambda i: (i,))
    h_spec = pl.BlockSpec((RBLK, H), lambda i: (i, 0))
    w_spec = pl.BlockSpec((H, H), lambda i: (0, 0))
    b_spec = pl.BlockSpec((1, H), lambda i: (0, 0))
    return pl.pallas_call(
        functools.partial(_sage_body, relu),
        grid=(GRID,),
        in_specs=[agg_spec, cnt_spec, h_spec, w_spec, b_spec, w_spec,
                  agg_spec, cnt_spec, h_spec, w_spec, b_spec, w_spec],
        out_specs=[h_spec, h_spec],
        out_shape=[jax.ShapeDtypeStruct((N, H), _f32)] * 2,
    )(aggf, cntf, hf, Wlf, blf, Wrf, aggm, cntm, hm, Wlm, blm, Wrm)


def _final_body(aggf, cntf, hf, Wlf, blf, Wrf,
                aggm, cntm, hm, Wlm, blm, Wrm,
                We1t, We1b, be1, oA, oB):
    def side(agg, cnt, h, Wl, bl, Wr):
        mean = agg / jnp.maximum(cnt, 1.0)[:, None]
        return (jnp.dot(mean, Wl[...], preferred_element_type=_f32) + bl[...]
                + jnp.dot(h[...], Wr[...], preferred_element_type=_f32))

    h_mft2 = side(aggf[...], cntf[...], hf, Wlf, blf, Wrf)
    h_mch2 = side(aggm[...], cntm[...], hm, Wlm, blm, Wrm)
    oA[...] = (jnp.dot(h_mch2, We1t[...], preferred_element_type=_f32)
               + be1[...]).astype(jnp.bfloat16)
    oB[...] = jnp.dot(h_mft2, We1b[...],
                      preferred_element_type=_f32).astype(jnp.bfloat16)


def _final_proj(aggf, cntf, hf, Wlf, blf, Wrf, aggm, cntm, hm, Wlm, blm, Wrm,
                We1t, We1b, be1):
    agg_spec = pl.BlockSpec((RBLK, H), lambda i: (i, 0))
    cnt_spec = pl.BlockSpec((RBLK,), lambda i: (i,))
    h_spec = pl.BlockSpec((RBLK, H), lambda i: (i, 0))
    w_spec = pl.BlockSpec((H, H), lambda i: (0, 0))
    b_spec = pl.BlockSpec((1, H), lambda i: (0, 0))
    return pl.pallas_call(
        _final_body,
        grid=(GRID,),
        in_specs=[agg_spec, cnt_spec, h_spec, w_spec, b_spec, w_spec,
                  agg_spec, cnt_spec, h_spec, w_spec, b_spec, w_spec,
                  w_spec, w_spec, b_spec],
        out_specs=[h_spec, h_spec],
        out_shape=[jax.ShapeDtypeStruct((N, H), jnp.bfloat16)] * 2,
    )(aggf, cntf, hf, Wlf, blf, Wrf, aggm, cntm, hm, Wlm, blm, Wrm,
      We1t, We1b, be1)


CHF = 80
NCHF = EPW // CHF


def _make_edge_scorer():
    mesh = plsc.VectorSubcoreMesh(core_axis_name="c", subcore_axis_name="s")
    out_type = jax.ShapeDtypeStruct((NW, EPW), _f32)
    scratch = [
        pltpu.VMEM((NCHF, CHF), jnp.int32),
        pltpu.VMEM((NCHF, CHF), jnp.int32),
        pltpu.VMEM((2, CHF, H), jnp.bfloat16),
        pltpu.VMEM((2, CHF, H), jnp.bfloat16),
        pltpu.VMEM((H,), jnp.bfloat16),
        pltpu.VMEM((16,), _f32),
        pltpu.VMEM((EPW,), _f32),
        pltpu.SemaphoreType.DMA,
        pltpu.SemaphoreType.DMA,
    ]

    def body(a_h, b_h, ei, w_h, be2_h, out_h,
             src_v, dst_v, a_rows, b_rows, w_v, be2_v, out_v, sem_a, sem_b):
        c = lax.axis_index("c")
        s = lax.axis_index("s")
        hs = pl.ds(c * NCHF, NCHF)
        pltpu.sync_copy(ei.at[0, s, hs], src_v)
        pltpu.sync_copy(ei.at[1, s, hs], dst_v)
        pltpu.sync_copy(w_h, w_v)
        pltpu.sync_copy(be2_h, be2_v)
        wq = []
        for q in range(H // 32):
            w0, w1 = plsc.unpack(w_v[pl.ds(q * 32, 32)],
                                 format=plsc.PackFormat.INTERLEAVED,
                                 preferred_element_type=_f32)
            wq.append((w0, w1))
        zero16 = jnp.zeros((16,), _f32)
        zero32 = jnp.zeros((32,), jnp.bfloat16)
        lane = lax.iota(jnp.int32, 16)

        def fire(j, bank):
            ca = pltpu.async_copy(a_h.at[src_v.at[j]], a_rows.at[bank], sem_a)
            cb = pltpu.async_copy(b_h.at[dst_v.at[j]], b_rows.at[bank], sem_b)
            return ca, cb

        fire(0, 0)

        def chunk(j, carry):
            bank = lax.rem(j, 2)

            @pl.when(j + 1 < NCHF)
            def _():
                fire(j + 1, lax.rem(j + 1, 2))

            pltpu.make_async_copy(a_h.at[src_v.at[j]], a_rows.at[bank],
                                  sem_a).wait()
            pltpu.make_async_copy(b_h.at[dst_v.at[j]], b_rows.at[bank],
                                  sem_b).wait()

            def group(g, carry2):
                vec = zero16
                for e in range(16):
                    idx = g * 16 + e
                    acc0 = zero16
                    acc1 = zero16
                    for q in range(H // 32):
                        va = a_rows[bank, idx, pl.ds(q * 32, 32)]
                        vb = b_rows[bank, idx, pl.ds(q * 32, 32)]
                        sp = jnp.maximum(va + vb, zero32)
                        s0, s1 = plsc.unpack(
                            sp, format=plsc.PackFormat.INTERLEAVED,
                            preferred_element_type=_f32)
                        acc0 = acc0 + s0 * wq[q][0]
                        acc1 = acc1 + s1 * wq[q][1]
                    vec = jnp.where(lane == e, jnp.sum(acc0 + acc1), vec)
                out_v[pl.ds(j * CHF + g * 16, 16)] = vec
                return carry2

            lax.fori_loop(0, CHF // 16, group, 0)
            return carry

        lax.fori_loop(0, NCHF, chunk, 0)
        bev = be2_v[...]

        def sig(i, carry):
            ix = pl.ds(i * 16, 16)
            v = out_v[ix] + bev
            out_v[ix] = 1.0 / (1.0 + jnp.exp(-v))
            return carry

        lax.fori_loop(0, EPW // 16, sig, 0)
        pltpu.sync_copy(out_v, out_h.at[2 * s + c])

    return pl.kernel(
        body, out_type=out_type, mesh=mesh, scratch_types=scratch,
        compiler_params=pltpu.CompilerParams(use_tc_tiling_on_sc=False, needs_layout_passes=False))


_edge_scorer = _make_edge_scorer()


def kernel(x_mch, x_mft, edge_index, W_mch, b_mch, W_mft, b_mft,
           Wl1_m2f, bl1_m2f, Wr1_m2f, Wl1_f2m, bl1_f2m, Wr1_f2m,
           Wl2_m2f, bl2_m2f, Wr2_m2f, Wl2_f2m, bl2_f2m, Wr2_f2m,
           We1, be1, We2, be2):
    ei = edge_index.astype(jnp.int32).reshape(2, NS, TPC, CH)
    z2d = jnp.zeros((RPT, H), _f32)
    z1d = jnp.zeros((N,), _f32)
    ones_h = jnp.ones((128,), _f32)

    h_mch, h_mft = _proj(x_mch, x_mft, W_mch, b_mch.reshape(1, H),
                         W_mft, b_mft.reshape(1, H))

    aggf1, aggm1, cntf, cntm = _agg_with_counts(
        h_mch, h_mft, ei, z2d, z1d, ones_h)

    h_mft1, h_mch1 = _sage_layer(
        True,
        aggf1, cntf, h_mft, Wl1_m2f, bl1_m2f.reshape(1, H), Wr1_m2f,
        aggm1, cntm, h_mch, Wl1_f2m, bl1_f2m.reshape(1, H), Wr1_f2m)

    aggf2, aggm2 = _agg_plain(h_mch1, h_mft1, ei, z2d)

    Aproj, Bproj = _final_proj(
        aggf2, cntf, h_mft1, Wl2_m2f, bl2_m2f.reshape(1, H), Wr2_m2f,
        aggm2, cntm, h_mch1, Wl2_f2m, bl2_f2m.reshape(1, H), Wr2_f2m,
        We1[:H], We1[H:], be1.reshape(1, H))

    out = _edge_scorer(Aproj, Bproj, ei,
                       We2[:, 0].astype(jnp.bfloat16),
                       jnp.broadcast_to(be2, (16,)).astype(_f32))
    return out.reshape(E, 1)


# speedup vs baseline: 1.0998x; 1.0637x over previous
import functools

import jax
import jax.numpy as jnp
from jax import lax
from jax.experimental import pallas as pl
from jax.experimental.pallas import tpu as pltpu
from jax.experimental.pallas import tpu_sc as plsc

N = 10000
E = 320000
DIN = 128
H = 64
NC = 2
NS = 16
NW = NC * NS
EPW = E // NW
CH = 80
NB = 5
N_PAD = 10240
RPT = N_PAD // NS
RBLK = 2048
GRID = N_PAD // RBLK

_f32 = jnp.float32


def _proj_body(xm, xf, Wm, bm, Wf, bf, hm, hf):
    hm[...] = jnp.dot(xm[...], Wm[...], preferred_element_type=_f32) + bm[...]
    hf[...] = jnp.dot(xf[...], Wf[...], preferred_element_type=_f32) + bf[...]


def _proj(x_mch, x_mft, W_mch, b_mch, W_mft, b_mft):
    return pl.pallas_call(
        _proj_body,
        grid=(GRID,),
        in_specs=[
            pl.BlockSpec((RBLK, DIN), lambda i: (i, 0)),
            pl.BlockSpec((RBLK, DIN), lambda i: (i, 0)),
            pl.BlockSpec((DIN, H), lambda i: (0, 0)),
            pl.BlockSpec((1, H), lambda i: (0, 0)),
            pl.BlockSpec((DIN, H), lambda i: (0, 0)),
            pl.BlockSpec((1, H), lambda i: (0, 0)),
        ],
        out_specs=[pl.BlockSpec((RBLK, H), lambda i: (i, 0))] * 2,
        out_shape=[jax.ShapeDtypeStruct((N, H), _f32)] * 2,
    )(x_mch, x_mft, W_mch, b_mch, W_mft, b_mft)


TPC = (E // NS) // CH


def _make_agg(with_counts):
    mesh = plsc.VectorSubcoreMesh(core_axis_name="c", subcore_axis_name="s")
    out_type = [
        jax.ShapeDtypeStruct((N_PAD, H), _f32),
        jax.ShapeDtypeStruct((N_PAD, H), _f32),
    ]
    scratch = [
        pltpu.VMEM((TPC // 2, CH), jnp.int32),
        pltpu.VMEM((TPC // 2, CH), jnp.int32),
        pltpu.VMEM((2, NB, CH, H), _f32),
        pltpu.VMEM_SHARED((N_PAD, H), _f32),
        pltpu.SemaphoreType.DMA,
        pltpu.SemaphoreType.DMA,
    ]
    if with_counts:
        out_type += [
            jax.ShapeDtypeStruct((N,), _f32),
            jax.ShapeDtypeStruct((N,), _f32),
        ]
        scratch += [
            pltpu.VMEM_SHARED((N,), _f32),
            pltpu.VMEM((128,), _f32),
            pltpu.SemaphoreType.DMA,
        ]

    def body(hm, hf, ei, z2d, *rest):
        if with_counts:
            (z1d, ones_h, o_aggf, o_aggm, o_cntf, o_cntm,
             src_v, dst_v, rows, acc, sem_g, sem_s,
             cnt, ones_v, sem_c) = rest
        else:
            (o_aggf, o_aggm,
             src_v, dst_v, rows, acc, sem_g, sem_s) = rest
        c = lax.axis_index("c")
        s = lax.axis_index("s")
        rs = pl.ds(s * RPT, RPT)
        pltpu.sync_copy(z2d, acc.at[rs])
        if with_counts:
            @pl.when(s == 0)
            def _():
                pltpu.sync_copy(z1d, cnt)

            pltpu.sync_copy(ones_h, ones_v)
        plsc.subcore_barrier()

        def sweep_half(tab, gidx, sidx):
            G = (TPC // 2) // NB

            def fire_g(g, bank):
                for b in range(NB):
                    pltpu.async_copy(tab.at[gidx.at[g * NB + b]],
                                     rows.at[bank, b], sem_g)

            def drain_g(g, bank):
                for b in range(NB):
                    pltpu.make_async_copy(tab.at[gidx.at[g * NB + b]],
                                          rows.at[bank, b], sem_g).wait()

            def fire_s(g, bank):
                for b in range(NB):
                    di = sidx.at[g * NB + b]
                    pltpu.async_copy(rows.at[bank, b], acc.at[di], sem_s,
                                     add=True)
                    if with_counts:
                        pltpu.async_copy(ones_v.at[pl.ds(0, CH)], cnt.at[di],
                                         sem_c, add=True)

            def drain_s(g, bank):
                for b in range(NB):
                    di = sidx.at[g * NB + b]
                    pltpu.make_async_copy(rows.at[bank, b], acc.at[di],
                                          sem_s).wait()
                    if with_counts:
                        pltpu.make_async_copy(ones_v.at[pl.ds(0, CH)],
                                              cnt.at[di], sem_c).wait()

            fire_g(0, 0)

            def step(g, carry):
                bank = lax.rem(g, 2)
                obank = 1 - bank

                @pl.when(g > 0)
                def _():
                    drain_s(g - 1, obank)

                @pl.when(g + 1 < G)
                def _():
                    fire_g(g + 1, obank)

                drain_g(g, bank)
                fire_s(g, bank)
                return carry

            lax.fori_loop(0, G, step, 0)
            drain_s(G - 1, (G - 1) % 2)

        for half in range(2):
            hs = pl.ds(half * (TPC // 2), TPC // 2)
            pltpu.sync_copy(ei.at[0, s, hs], src_v)
            pltpu.sync_copy(ei.at[1, s, hs], dst_v)

            @pl.when(c == 0)
            def _():
                sweep_half(hm, src_v, dst_v)

            @pl.when(c == 1)
            def _():
                sweep_half(hf, dst_v, src_v)

        plsc.subcore_barrier()

        @pl.when(c == 0)
        def _():
            pltpu.sync_copy(acc.at[rs], o_aggf.at[rs])

        @pl.when(c == 1)
        def _():
            pltpu.sync_copy(acc.at[rs], o_aggm.at[rs])

        if with_counts:
            @pl.when((c == 0) & (s == 0))
            def _():
                pltpu.sync_copy(cnt, o_cntf)

            @pl.when((c == 1) & (s == 0))
            def _():
                pltpu.sync_copy(cnt, o_cntm)

    return pl.kernel(
        body, out_type=out_type, mesh=mesh, scratch_types=scratch,
        compiler_params=pltpu.CompilerParams(use_tc_tiling_on_sc=False, needs_layout_passes=False))


_agg_with_counts = _make_agg(True)
_agg_plain = _make_agg(False)


def _sage_body(relu, aggf, cntf, hf, Wlf, blf, Wrf,
               aggm, cntm, hm, Wlm, blm, Wrm, of, om):
    def side(agg, cnt, h, Wl, bl, Wr):
        mean = agg / jnp.maximum(cnt, 1.0)[:, None]
        r = (jnp.dot(mean, Wl[...], preferred_element_type=_f32) + bl[...]
             + jnp.dot(h[...], Wr[...], preferred_element_type=_f32))
        return jnp.maximum(r, 0.0) if relu else r

    of[...] = side(aggf[...], cntf[...], hf, Wlf, blf, Wrf)
    om[...] = side(aggm[...], cntm[...], hm, Wlm, blm, Wrm)


def _sage_layer(relu, aggf, cntf, hf, Wlf, blf, Wrf, aggm, cntm, hm, Wlm, blm, Wrm):
    agg_spec = pl.BlockSpec((RBLK, H), lambda i: (i, 0))
    cnt_spec = pl.BlockSpec((RBLK,), lambda i: (i,))
    h_spec = pl.BlockSpec((RBLK, H), lambda i: (i, 0))
    w_spec = pl.BlockSpec((H, H), lambda i: (0, 0))
    b_spec = pl.BlockSpec((1, H), lambda i: (0, 0))
    return pl.pallas_call(
        functools.partial(_sage_body, relu),
        grid=(GRID,),
        in_specs=[agg_spec, cnt_spec, h_spec, w_spec, b_spec, w_spec,
                  agg_spec, cnt_spec, h_spec, w_spec, b_spec, w_spec],
        out_specs=[h_spec, h_spec],
        out_shape=[jax.ShapeDtypeStruct((N, H), _f32)] * 2,
    )(aggf, cntf, hf, Wlf, blf, Wrf, aggm, cntm, hm, Wlm, blm, Wrm)


def _final_body(aggf, cntf, hf, Wlf, blf, Wrf,
                aggm, cntm, hm, Wlm, blm, Wrm,
                We1t, We1b, be1, oA, oB):
    def side(agg, cnt, h, Wl, bl, Wr):
        mean = agg / jnp.maximum(cnt, 1.0)[:, None]
        return (jnp.dot(mean, Wl[...], preferred_element_type=_f32) + bl[...]
                + jnp.dot(h[...], Wr[...], preferred_element_type=_f32))

    h_mft2 = side(aggf[...], cntf[...], hf, Wlf, blf, Wrf)
    h_mch2 = side(aggm[...], cntm[...], hm, Wlm, blm, Wrm)
    oA[...] = (jnp.dot(h_mch2, We1t[...], preferred_element_type=_f32)
               + be1[...]).astype(jnp.bfloat16)
    oB[...] = jnp.dot(h_mft2, We1b[...],
                      preferred_element_type=_f32).astype(jnp.bfloat16)


def _final_proj(aggf, cntf, hf, Wlf, blf, Wrf, aggm, cntm, hm, Wlm, blm, Wrm,
                We1t, We1b, be1):
    agg_spec = pl.BlockSpec((RBLK, H), lambda i: (i, 0))
    cnt_spec = pl.BlockSpec((RBLK,), lambda i: (i,))
    h_spec = pl.BlockSpec((RBLK, H), lambda i: (i, 0))
    w_spec = pl.BlockSpec((H, H), lambda i: (0, 0))
    b_spec = pl.BlockSpec((1, H), lambda i: (0, 0))
    return pl.pallas_call(
        _final_body,
        grid=(GRID,),
        in_specs=[agg_spec, cnt_spec, h_spec, w_spec, b_spec, w_spec,
                  agg_spec, cnt_spec, h_spec, w_spec, b_spec, w_spec,
                  w_spec, w_spec, b_spec],
        out_specs=[h_spec, h_spec],
        out_shape=[jax.ShapeDtypeStruct((N, H), jnp.bfloat16)] * 2,
    )(aggf, cntf, hf, Wlf, blf, Wrf, aggm, cntm, hm, Wlm, blm, Wrm,
      We1t, We1b, be1)


CHF = 80
NCHF = EPW // CHF


def _make_edge_scorer():
    mesh = plsc.VectorSubcoreMesh(core_axis_name="c", subcore_axis_name="s")
    out_type = jax.ShapeDtypeStruct((NW, EPW), _f32)
    scratch = [
        pltpu.VMEM((NCHF, CHF), jnp.int32),
        pltpu.VMEM((NCHF, CHF), jnp.int32),
        pltpu.VMEM((4, CHF, H), jnp.bfloat16),
        pltpu.VMEM((4, CHF, H), jnp.bfloat16),
        pltpu.VMEM((H,), jnp.bfloat16),
        pltpu.VMEM((16,), _f32),
        pltpu.VMEM((EPW,), _f32),
        pltpu.SemaphoreType.DMA,
        pltpu.SemaphoreType.DMA,
    ]

    def body(a_h, b_h, ei, w_h, be2_h, out_h,
             src_v, dst_v, a_rows, b_rows, w_v, be2_v, out_v, sem_a, sem_b):
        c = lax.axis_index("c")
        s = lax.axis_index("s")
        hs = pl.ds(c * NCHF, NCHF)
        pltpu.sync_copy(ei.at[0, s, hs], src_v)
        pltpu.sync_copy(ei.at[1, s, hs], dst_v)
        pltpu.sync_copy(w_h, w_v)
        pltpu.sync_copy(be2_h, be2_v)
        wq = []
        for q in range(H // 32):
            w0, w1 = plsc.unpack(w_v[pl.ds(q * 32, 32)],
                                 format=plsc.PackFormat.INTERLEAVED,
                                 preferred_element_type=_f32)
            wq.append((w0, w1))
        zero16 = jnp.zeros((16,), _f32)
        zero32 = jnp.zeros((32,), jnp.bfloat16)
        lane = lax.iota(jnp.int32, 16)

        def fire(j, bank):
            ca = pltpu.async_copy(a_h.at[src_v.at[j]], a_rows.at[bank], sem_a)
            cb = pltpu.async_copy(b_h.at[dst_v.at[j]], b_rows.at[bank], sem_b)
            return ca, cb

        for p in range(3):
            fire(p, p)

        def chunk(j, carry):
            bank = lax.rem(j, 4)

            @pl.when(j + 3 < NCHF)
            def _():
                fire(j + 3, lax.rem(j + 3, 4))

            pltpu.make_async_copy(a_h.at[src_v.at[j]], a_rows.at[bank],
                                  sem_a).wait()
            pltpu.make_async_copy(b_h.at[dst_v.at[j]], b_rows.at[bank],
                                  sem_b).wait()

            for g in range(CHF // 16):
                vec = zero16
                for e in range(16):
                    idx = g * 16 + e
                    acc0 = zero16
                    acc1 = zero16
                    for q in range(H // 32):
                        va = a_rows[bank, idx, pl.ds(q * 32, 32)]
                        vb = b_rows[bank, idx, pl.ds(q * 32, 32)]
                        sp = jnp.maximum(va + vb, zero32)
                        s0, s1 = plsc.unpack(
                            sp, format=plsc.PackFormat.INTERLEAVED,
                            preferred_element_type=_f32)
                        acc0 = acc0 + s0 * wq[q][0]
                        acc1 = acc1 + s1 * wq[q][1]
                    vec = jnp.where(lane == e, jnp.sum(acc0 + acc1), vec)
                out_v[pl.ds(j * CHF + g * 16, 16)] = vec
            return carry

        lax.fori_loop(0, NCHF, chunk, 0)
        bev = be2_v[...]

        def sig(i, carry):
            ix = pl.ds(i * 16, 16)
            v = out_v[ix] + bev
            out_v[ix] = 1.0 / (1.0 + jnp.exp(-v))
            return carry

        lax.fori_loop(0, EPW // 16, sig, 0)
        pltpu.sync_copy(out_v, out_h.at[2 * s + c])

    return pl.kernel(
        body, out_type=out_type, mesh=mesh, scratch_types=scratch,
        compiler_params=pltpu.CompilerParams(use_tc_tiling_on_sc=False, needs_layout_passes=False))


_edge_scorer = _make_edge_scorer()


def kernel(x_mch, x_mft, edge_index, W_mch, b_mch, W_mft, b_mft,
           Wl1_m2f, bl1_m2f, Wr1_m2f, Wl1_f2m, bl1_f2m, Wr1_f2m,
           Wl2_m2f, bl2_m2f, Wr2_m2f, Wl2_f2m, bl2_f2m, Wr2_f2m,
           We1, be1, We2, be2):
    ei = edge_index.astype(jnp.int32).reshape(2, NS, TPC, CH)
    z2d = jnp.zeros((RPT, H), _f32)
    z1d = jnp.zeros((N,), _f32)
    ones_h = jnp.ones((128,), _f32)

    h_mch, h_mft = _proj(x_mch, x_mft, W_mch, b_mch.reshape(1, H),
                         W_mft, b_mft.reshape(1, H))

    aggf1, aggm1, cntf, cntm = _agg_with_counts(
        h_mch, h_mft, ei, z2d, z1d, ones_h)

    h_mft1, h_mch1 = _sage_layer(
        True,
        aggf1, cntf, h_mft, Wl1_m2f, bl1_m2f.reshape(1, H), Wr1_m2f,
        aggm1, cntm, h_mch, Wl1_f2m, bl1_f2m.reshape(1, H), Wr1_f2m)

    aggf2, aggm2 = _agg_plain(h_mch1, h_mft1, ei, z2d)

    Aproj, Bproj = _final_proj(
        aggf2, cntf, h_mft1, Wl2_m2f, bl2_m2f.reshape(1, H), Wr2_m2f,
        aggm2, cntm, h_mch1, Wl2_f2m, bl2_f2m.reshape(1, H), Wr2_f2m,
        We1[:H], We1[H:], be1.reshape(1, H))

    out = _edge_scorer(Aproj, Bproj, ei,
                       We2[:, 0].astype(jnp.bfloat16),
                       jnp.broadcast_to(be2, (16,)).astype(_f32))
    return out.reshape(E, 1)


# speedup vs baseline: 1.1172x; 1.0158x over previous
import functools

import jax
import jax.numpy as jnp
from jax import lax
from jax.experimental import pallas as pl
from jax.experimental.pallas import tpu as pltpu
from jax.experimental.pallas import tpu_sc as plsc

N = 10000
E = 320000
DIN = 128
H = 64
NC = 2
NS = 16
NW = NC * NS
EPW = E // NW
CH = 80
NB = 5
N_PAD = 10240
RPT = N_PAD // NS
RBLK = 5120
GRID = N_PAD // RBLK

_f32 = jnp.float32


def _proj_body(xm, xf, Wm, bm, Wf, bf, hm, hf):
    hm[...] = jnp.dot(xm[...], Wm[...], preferred_element_type=_f32) + bm[...]
    hf[...] = jnp.dot(xf[...], Wf[...], preferred_element_type=_f32) + bf[...]


def _proj(x_mch, x_mft, W_mch, b_mch, W_mft, b_mft):
    return pl.pallas_call(
        _proj_body,
        grid=(GRID,),
        in_specs=[
            pl.BlockSpec((RBLK, DIN), lambda i: (i, 0)),
            pl.BlockSpec((RBLK, DIN), lambda i: (i, 0)),
            pl.BlockSpec((DIN, H), lambda i: (0, 0)),
            pl.BlockSpec((1, H), lambda i: (0, 0)),
            pl.BlockSpec((DIN, H), lambda i: (0, 0)),
            pl.BlockSpec((1, H), lambda i: (0, 0)),
        ],
        out_specs=[pl.BlockSpec((RBLK, H), lambda i: (i, 0))] * 2,
        out_shape=[jax.ShapeDtypeStruct((N, H), _f32)] * 2,
    )(x_mch, x_mft, W_mch, b_mch, W_mft, b_mft)


TPC = (E // NS) // CH


def _make_agg(with_counts):
    mesh = plsc.VectorSubcoreMesh(core_axis_name="c", subcore_axis_name="s")
    out_type = [
        jax.ShapeDtypeStruct((N_PAD, H), _f32),
        jax.ShapeDtypeStruct((N_PAD, H), _f32),
    ]
    scratch = [
        pltpu.VMEM((TPC // 2, CH), jnp.int32),
        pltpu.VMEM((TPC // 2, CH), jnp.int32),
        pltpu.VMEM((2, NB, CH, H), _f32),
        pltpu.VMEM_SHARED((N_PAD, H), _f32),
        pltpu.SemaphoreType.DMA,
        pltpu.SemaphoreType.DMA,
    ]
    if with_counts:
        out_type += [
            jax.ShapeDtypeStruct((N,), _f32),
            jax.ShapeDtypeStruct((N,), _f32),
        ]
        scratch += [
            pltpu.VMEM_SHARED((N,), _f32),
            pltpu.VMEM((128,), _f32),
            pltpu.SemaphoreType.DMA,
        ]

    def body(hm, hf, ei, z2d, *rest):
        if with_counts:
            (z1d, ones_h, o_aggf, o_aggm, o_cntf, o_cntm,
             src_v, dst_v, rows, acc, sem_g, sem_s,
             cnt, ones_v, sem_c) = rest
        else:
            (o_aggf, o_aggm,
             src_v, dst_v, rows, acc, sem_g, sem_s) = rest
        c = lax.axis_index("c")
        s = lax.axis_index("s")
        rs = pl.ds(s * RPT, RPT)
        pltpu.sync_copy(z2d, acc.at[rs])
        if with_counts:
            @pl.when(s == 0)
            def _():
                pltpu.sync_copy(z1d, cnt)

            pltpu.sync_copy(ones_h, ones_v)
        plsc.subcore_barrier()

        def sweep_half(tab, gidx, sidx):
            G = (TPC // 2) // NB

            def fire_g(g, bank):
                for b in range(NB):
                    pltpu.async_copy(tab.at[gidx.at[g * NB + b]],
                                     rows.at[bank, b], sem_g)

            def drain_g(g, bank):
                for b in range(NB):
                    pltpu.make_async_copy(tab.at[gidx.at[g * NB + b]],
                                          rows.at[bank, b], sem_g).wait()

            def fire_s(g, bank):
                for b in range(NB):
                    di = sidx.at[g * NB + b]
                    pltpu.async_copy(rows.at[bank, b], acc.at[di], sem_s,
                                     add=True)
                    if with_counts:
                        pltpu.async_copy(ones_v.at[pl.ds(0, CH)], cnt.at[di],
                                         sem_c, add=True)

            def drain_s(g, bank):
                for b in range(NB):
                    di = sidx.at[g * NB + b]
                    pltpu.make_async_copy(rows.at[bank, b], acc.at[di],
                                          sem_s).wait()
                    if with_counts:
                        pltpu.make_async_copy(ones_v.at[pl.ds(0, CH)],
                                              cnt.at[di], sem_c).wait()

            fire_g(0, 0)

            def step(g, carry):
                bank = lax.rem(g, 2)
                obank = 1 - bank

                @pl.when(g > 0)
                def _():
                    drain_s(g - 1, obank)

                @pl.when(g + 1 < G)
                def _():
                    fire_g(g + 1, obank)

                drain_g(g, bank)
                fire_s(g, bank)
                return carry

            lax.fori_loop(0, G, step, 0)
            drain_s(G - 1, (G - 1) % 2)

        for half in range(2):
            hs = pl.ds(half * (TPC // 2), TPC // 2)
            pltpu.sync_copy(ei.at[0, s, hs], src_v)
            pltpu.sync_copy(ei.at[1, s, hs], dst_v)

            @pl.when(c == 0)
            def _():
                sweep_half(hm, src_v, dst_v)

            @pl.when(c == 1)
            def _():
                sweep_half(hf, dst_v, src_v)

        plsc.subcore_barrier()

        @pl.when(c == 0)
        def _():
            pltpu.sync_copy(acc.at[rs], o_aggf.at[rs])

        @pl.when(c == 1)
        def _():
            pltpu.sync_copy(acc.at[rs], o_aggm.at[rs])

        if with_counts:
            @pl.when((c == 0) & (s == 0))
            def _():
                pltpu.sync_copy(cnt, o_cntf)

            @pl.when((c == 1) & (s == 0))
            def _():
                pltpu.sync_copy(cnt, o_cntm)

    return pl.kernel(
        body, out_type=out_type, mesh=mesh, scratch_types=scratch,
        compiler_params=pltpu.CompilerParams(use_tc_tiling_on_sc=False, needs_layout_passes=False))


_agg_with_counts = _make_agg(True)
_agg_plain = _make_agg(False)


def _sage_body(relu, aggf, cntf, hf, Wlf, blf, Wrf,
               aggm, cntm, hm, Wlm, blm, Wrm, of, om):
    def side(agg, cnt, h, Wl, bl, Wr):
        mean = agg / jnp.maximum(cnt, 1.0)[:, None]
        r = (jnp.dot(mean, Wl[...], preferred_element_type=_f32) + bl[...]
             + jnp.dot(h[...], Wr[...], preferred_element_type=_f32))
        return jnp.maximum(r, 0.0) if relu else r

    of[...] = side(aggf[...], cntf[...], hf, Wlf, blf, Wrf)
    om[...] = side(aggm[...], cntm[...], hm, Wlm, blm, Wrm)


def _sage_layer(relu, aggf, cntf, hf, Wlf, blf, Wrf, aggm, cntm, hm, Wlm, blm, Wrm):
    agg_spec = pl.BlockSpec((RBLK, H), lambda i: (i, 0))
    cnt_spec = pl.BlockSpec((RBLK,), lambda i: (i,))
    h_spec = pl.BlockSpec((RBLK, H), lambda i: (i, 0))
    w_spec = pl.BlockSpec((H, H), lambda i: (0, 0))
    b_spec = pl.BlockSpec((1, H), lambda i: (0, 0))
    return pl.pallas_call(
        functools.partial(_sage_body, relu),
        grid=(GRID,),
        in_specs=[agg_spec, cnt_spec, h_spec, w_spec, b_spec, w_spec,
                  agg_spec, cnt_spec, h_spec, w_spec, b_spec, w_spec],
        out_specs=[h_spec, h_spec],
        out_shape=[jax.ShapeDtypeStruct((N, H), _f32)] * 2,
    )(aggf, cntf, hf, Wlf, blf, Wrf, aggm, cntm, hm, Wlm, blm, Wrm)


def _final_body(aggf, cntf, hf, Wlf, blf, Wrf,
                aggm, cntm, hm, Wlm, blm, Wrm,
                We1t, We1b, be1, oA, oB):
    def side(agg, cnt, h, Wl, bl, Wr):
        mean = agg / jnp.maximum(cnt, 1.0)[:, None]
        return (jnp.dot(mean, Wl[...], preferred_element_type=_f32) + bl[...]
                + jnp.dot(h[...], Wr[...], preferred_element_type=_f32))

    h_mft2 = side(aggf[...], cntf[...], hf, Wlf, blf, Wrf)
    h_mch2 = side(aggm[...], cntm[...], hm, Wlm, blm, Wrm)
    oA[...] = (jnp.dot(h_mch2, We1t[...], preferred_element_type=_f32)
               + be1[...]).astype(jnp.bfloat16)
    oB[...] = jnp.dot(h_mft2, We1b[...],
                      preferred_element_type=_f32).astype(jnp.bfloat16)


def _final_proj(aggf, cntf, hf, Wlf, blf, Wrf, aggm, cntm, hm, Wlm, blm, Wrm,
                We1t, We1b, be1):
    agg_spec = pl.BlockSpec((RBLK, H), lambda i: (i, 0))
    cnt_spec = pl.BlockSpec((RBLK,), lambda i: (i,))
    h_spec = pl.BlockSpec((RBLK, H), lambda i: (i, 0))
    w_spec = pl.BlockSpec((H, H), lambda i: (0, 0))
    b_spec = pl.BlockSpec((1, H), lambda i: (0, 0))
    return pl.pallas_call(
        _final_body,
        grid=(GRID,),
        in_specs=[agg_spec, cnt_spec, h_spec, w_spec, b_spec, w_spec,
                  agg_spec, cnt_spec, h_spec, w_spec, b_spec, w_spec,
                  w_spec, w_spec, b_spec],
        out_specs=[h_spec, h_spec],
        out_shape=[jax.ShapeDtypeStruct((N, H), jnp.bfloat16)] * 2,
    )(aggf, cntf, hf, Wlf, blf, Wrf, aggm, cntm, hm, Wlm, blm, Wrm,
      We1t, We1b, be1)


CHF = 80
NCHF = EPW // CHF


def _make_edge_scorer():
    mesh = plsc.VectorSubcoreMesh(core_axis_name="c", subcore_axis_name="s")
    out_type = jax.ShapeDtypeStruct((NW, EPW), _f32)
    scratch = [
        pltpu.VMEM((NCHF, CHF), jnp.int32),
        pltpu.VMEM((NCHF, CHF), jnp.int32),
        pltpu.VMEM((4, CHF, H), jnp.bfloat16),
        pltpu.VMEM((4, CHF, H), jnp.bfloat16),
        pltpu.VMEM((H,), jnp.bfloat16),
        pltpu.VMEM((16,), _f32),
        pltpu.VMEM((EPW,), _f32),
        pltpu.SemaphoreType.DMA,
        pltpu.SemaphoreType.DMA,
    ]

    def body(a_h, b_h, ei, w_h, be2_h, out_h,
             src_v, dst_v, a_rows, b_rows, w_v, be2_v, out_v, sem_a, sem_b):
        c = lax.axis_index("c")
        s = lax.axis_index("s")
        hs = pl.ds(c * NCHF, NCHF)
        pltpu.sync_copy(ei.at[0, s, hs], src_v)
        pltpu.sync_copy(ei.at[1, s, hs], dst_v)
        pltpu.sync_copy(w_h, w_v)
        pltpu.sync_copy(be2_h, be2_v)
        wq = []
        for q in range(H // 32):
            w0, w1 = plsc.unpack(w_v[pl.ds(q * 32, 32)],
                                 format=plsc.PackFormat.INTERLEAVED,
                                 preferred_element_type=_f32)
            wq.append((w0, w1))
        zero16 = jnp.zeros((16,), _f32)
        zero32 = jnp.zeros((32,), jnp.bfloat16)
        lane = lax.iota(jnp.int32, 16)

        def fire(j, bank):
            ca = pltpu.async_copy(a_h.at[src_v.at[j]], a_rows.at[bank], sem_a)
            cb = pltpu.async_copy(b_h.at[dst_v.at[j]], b_rows.at[bank], sem_b)
            return ca, cb

        for p in range(3):
            fire(p, p)

        def chunk(j, carry):
            bank = lax.rem(j, 4)

            @pl.when(j + 3 < NCHF)
            def _():
                fire(j + 3, lax.rem(j + 3, 4))

            pltpu.make_async_copy(a_h.at[src_v.at[j]], a_rows.at[bank],
                                  sem_a).wait()
            pltpu.make_async_copy(b_h.at[dst_v.at[j]], b_rows.at[bank],
                                  sem_b).wait()

            for g in range(CHF // 16):
                vec = zero16
                for e in range(16):
                    idx = g * 16 + e
                    acc0 = zero16
                    acc1 = zero16
                    for q in range(H // 32):
                        va = a_rows[bank, idx, pl.ds(q * 32, 32)]
                        vb = b_rows[bank, idx, pl.ds(q * 32, 32)]
                        sp = jnp.maximum(va + vb, zero32)
                        s0, s1 = plsc.unpack(
                            sp, format=plsc.PackFormat.INTERLEAVED,
                            preferred_element_type=_f32)
                        acc0 = acc0 + s0 * wq[q][0]
                        acc1 = acc1 + s1 * wq[q][1]
                    vec = jnp.where(lane == e, jnp.sum(acc0 + acc1), vec)
                out_v[pl.ds(j * CHF + g * 16, 16)] = vec
            return carry

        lax.fori_loop(0, NCHF, chunk, 0)
        bev = be2_v[...]

        def sig(i, carry):
            ix = pl.ds(i * 16, 16)
            v = out_v[ix] + bev
            out_v[ix] = 1.0 / (1.0 + jnp.exp(-v))
            return carry

        lax.fori_loop(0, EPW // 16, sig, 0)
        pltpu.sync_copy(out_v, out_h.at[2 * s + c])

    return pl.kernel(
        body, out_type=out_type, mesh=mesh, scratch_types=scratch,
        compiler_params=pltpu.CompilerParams(use_tc_tiling_on_sc=False, needs_layout_passes=False))


_edge_scorer = _make_edge_scorer()


def kernel(x_mch, x_mft, edge_index, W_mch, b_mch, W_mft, b_mft,
           Wl1_m2f, bl1_m2f, Wr1_m2f, Wl1_f2m, bl1_f2m, Wr1_f2m,
           Wl2_m2f, bl2_m2f, Wr2_m2f, Wl2_f2m, bl2_f2m, Wr2_f2m,
           We1, be1, We2, be2):
    ei = edge_index.astype(jnp.int32).reshape(2, NS, TPC, CH)
    z2d = jnp.zeros((RPT, H), _f32)
    z1d = jnp.zeros((N,), _f32)
    ones_h = jnp.ones((128,), _f32)

    h_mch, h_mft = _proj(x_mch, x_mft, W_mch, b_mch.reshape(1, H),
                         W_mft, b_mft.reshape(1, H))

    aggf1, aggm1, cntf, cntm = _agg_with_counts(
        h_mch, h_mft, ei, z2d, z1d, ones_h)

    h_mft1, h_mch1 = _sage_layer(
        True,
        aggf1, cntf, h_mft, Wl1_m2f, bl1_m2f.reshape(1, H), Wr1_m2f,
        aggm1, cntm, h_mch, Wl1_f2m, bl1_f2m.reshape(1, H), Wr1_f2m)

    aggf2, aggm2 = _agg_plain(h_mch1, h_mft1, ei, z2d)

    Aproj, Bproj = _final_proj(
        aggf2, cntf, h_mft1, Wl2_m2f, bl2_m2f.reshape(1, H), Wr2_m2f,
        aggm2, cntm, h_mch1, Wl2_f2m, bl2_f2m.reshape(1, H), Wr2_f2m,
        We1[:H], We1[H:], be1.reshape(1, H))

    out = _edge_scorer(Aproj, Bproj, ei,
                       We2[:, 0].astype(jnp.bfloat16),
                       jnp.broadcast_to(be2, (16,)).astype(_f32))
    return out.reshape(E, 1)
